# Initial kernel scaffold; baseline (speedup 1.0000x reference)
import jax
import jax.numpy as jnp
from jax.experimental import pallas as pl


def kernel(ref, hyp):
    raise NotImplementedError("write your pallas kernel here")


# speedup vs baseline: 5.5769x; 5.5769x over previous
import functools

import jax
import jax.numpy as jnp
from jax import lax
from jax.experimental import pallas as pl
from jax.experimental.pallas import tpu as pltpu
from jax.experimental.pallas import tpu_sc as plsc

R = 2048
H = 2048
B = 16
NSUB = 16
ROWS = R // NSUB
CB = 32
NB = H // CB
STEPS = NB + NSUB - 1


def _body(ref_hbm, hyp_hbm, out_hbm, ref_v, hyp_v, row_v, bnd_in, bnd_out,
          corner_v, spmem):
    cid = lax.axis_index("c")
    sid = lax.axis_index("s")

    pltpu.sync_copy(ref_hbm.at[pl.ds(sid * (ROWS * B), ROWS * B)], ref_v)
    pltpu.sync_copy(hyp_hbm, hyp_v)

    base_m1 = (sid * ROWS).astype(jnp.float32)
    zero_vec = jnp.zeros((B,), jnp.float32)
    corner_v[...] = zero_vec + base_m1

    def init_row(r, _):
        row_v[pl.ds(r * B, B)] = zero_vec + (base_m1 + 1.0
                                             + r.astype(jnp.float32))
        return 0

    lax.fori_loop(0, ROWS, init_row, 0)

    def step(k, _):
        b = k - sid
        valid = jnp.logical_and(b >= 0, b < NB)

        @pl.when(jnp.logical_and(valid, sid > 0))
        def _():
            slot = ((k + 1) % 2) * NSUB + (sid - 1)
            pltpu.sync_copy(spmem.at[pl.ds(slot * (CB * B), CB * B)], bnd_in)

        @pl.when(jnp.logical_and(valid, sid == 0))
        def _():
            def fill(jj, _):
                bnd_in[pl.ds(jj * B, B)] = (
                    zero_vec + (b * CB + jj + 1).astype(jnp.float32))
                return 0
            lax.fori_loop(0, CB, fill, 0)

        @pl.when(valid)
        def _():
            def col_body(jj, corner):
                j = b * CB + jj
                hv = hyp_v[pl.ds(j * B, B)]
                topv = bnd_in[pl.ds(jj * B, B)]

                def cell(r, carry):
                    left, diag = carry
                    prev = row_v[pl.ds(r * B, B)]
                    c = jnp.where(ref_v[pl.ds(r * B, B)] == hv, 0.0, 1.0)
                    newv = jnp.minimum(jnp.minimum(prev, left) + 1.0,
                                       diag + c)
                    row_v[pl.ds(r * B, B)] = newv
                    return newv, prev

                left, _unused = lax.fori_loop(0, ROWS, cell, (topv, corner))
                bnd_out[pl.ds(jj * B, B)] = left
                return topv

            corner = lax.fori_loop(0, CB, col_body, corner_v[...])
            corner_v[...] = corner
            slot = (k % 2) * NSUB + sid
            pltpu.sync_copy(bnd_out, spmem.at[pl.ds(slot * (CB * B), CB * B)])

        plsc.subcore_barrier()
        return 0

    lax.fori_loop(0, STEPS, step, 0)

    @pl.when(jnp.logical_and(cid == 0, sid == NSUB - 1))
    def _():
        pltpu.sync_copy(bnd_out.at[pl.ds((CB - 1) * B, B)], out_hbm)


@jax.jit
def kernel(ref, hyp):
    mesh = plsc.VectorSubcoreMesh(core_axis_name="c", subcore_axis_name="s")
    f = functools.partial(
        pl.kernel,
        mesh=mesh,
        out_type=jax.ShapeDtypeStruct((B,), jnp.float32),
        scratch_types=[
            pltpu.VMEM((ROWS * B,), jnp.int32),
            pltpu.VMEM((H * B,), jnp.int32),
            pltpu.VMEM((ROWS * B,), jnp.float32),
            pltpu.VMEM((CB * B,), jnp.float32),
            pltpu.VMEM((CB * B,), jnp.float32),
            pltpu.VMEM((B,), jnp.float32),
            pltpu.VMEM_SHARED((2 * NSUB * CB * B,), jnp.float32),
        ],
    )(_body)
    return f(ref.reshape(R * B), hyp.reshape(H * B))
